# baseline (speedup 1.0000x reference)
import functools

import jax
import jax.numpy as jnp
from jax import lax
from jax.experimental import pallas as pl
from jax.experimental.pallas import tpu as pltpu
from jax.experimental.pallas import tpu_sc as plsc

_NDIM = 6
_H = 10
_NUM_WORKERS = 32
_IDX_MINOR = 128


def _tc_index_body(states_t_ref, idx_ref, pair_ref):
    x = states_t_ref[...]
    c = jnp.clip(jnp.floor(x * float(_H)), 0.0, float(_H - 1)).astype(jnp.int32)
    powers = (_H ** jnp.arange(_NDIM, dtype=jnp.int32)).reshape(_NDIM, 1)
    idx = jnp.sum(c * powers, axis=0)
    idx_ref[...] = idx
    pair_ref[...] = idx >> 1


def _make_index_kernel(batch):
    return pl.pallas_call(
        _tc_index_body,
        out_shape=(
            jax.ShapeDtypeStruct((batch,), jnp.int32),
            jax.ShapeDtypeStruct((batch,), jnp.int32),
        ),
    )


def _make_sc_gather(batch, n_pairs):
    b_per_w = batch // _NUM_WORKERS
    n_idx_rows = b_per_w // _IDX_MINOR
    mesh = plsc.VectorSubcoreMesh(core_axis_name="c", subcore_axis_name="s")

    @functools.partial(
        pl.kernel,
        mesh=mesh,
        compiler_params=pltpu.CompilerParams(use_tc_tiling_on_sc=True),
        out_type=jax.ShapeDtypeStruct((batch, 128), jnp.float32),
        scratch_types=[
            [pltpu.VMEM((1, _IDX_MINOR), jnp.int32) for _ in range(4)],
            pltpu.VMEM((b_per_w, 128), jnp.float32),
            pltpu.SemaphoreType.DMA,
        ],
    )
    def sc_gather(pair_hbm, table_hbm, out_hbm, idx_vs, rows_v, sem):
        wid = lax.axis_index("s") * 2 + lax.axis_index("c")
        base = wid * b_per_w
        for k in range(n_idx_rows):
            pltpu.sync_copy(pair_hbm.at[wid, k], idx_vs[k])
        copies = []
        for k in range(n_idx_rows):
            cp = pltpu.make_async_copy(
                table_hbm.at[idx_vs[k].at[0]],
                rows_v.at[pl.ds(k * _IDX_MINOR, _IDX_MINOR)],
                sem,
            )
            cp.start()
            copies.append(cp)
        for cp in copies:
            cp.wait()
        pltpu.sync_copy(rows_v, out_hbm.at[pl.ds(base, b_per_w)])

    return sc_gather


def _tc_select_body(rows_ref, idx_ref, out_ref):
    x = rows_ref[...]
    par = (idx_ref[...] & 1).astype(jnp.bool_)
    lo = x[:, :64]
    hi = x[:, 64:]
    out_ref[...] = jnp.where(par, hi, lo)


def _make_select_kernel(batch, out_dim):
    return pl.pallas_call(
        _tc_select_body,
        out_shape=jax.ShapeDtypeStruct((batch, out_dim), jnp.float32),
    )


def kernel(preprocessed_states, table):
    batch = preprocessed_states.shape[0]
    n_states, out_dim = table.shape
    n_pairs = n_states * out_dim // 128
    b_per_w = batch // _NUM_WORKERS
    idx, pair = _make_index_kernel(batch)(preprocessed_states.T)
    pair4d = pair.reshape(_NUM_WORKERS, b_per_w // _IDX_MINOR, 1, _IDX_MINOR)
    table2 = table.reshape(n_pairs, 128)
    rows = _make_sc_gather(batch, n_pairs)(pair4d, table2)
    return _make_select_kernel(batch, out_dim)(rows, idx.reshape(batch, 1))


# speedup vs baseline: 1.6098x; 1.6098x over previous
import functools

import jax
import jax.numpy as jnp
from jax import lax
from jax.experimental import pallas as pl
from jax.experimental.pallas import tpu as pltpu
from jax.experimental.pallas import tpu_sc as plsc

_NDIM = 6
_H = 10
_NUM_WORKERS = 32
_CHUNK = 128
_GRP = 8
_LANES = 16


def _tc_index_body(states_t_ref, idx_ref):
    x = states_t_ref[...]
    c = jnp.clip(jnp.floor(x * float(_H)), 0.0, float(_H - 1)).astype(jnp.int32)
    powers = (_H ** jnp.arange(_NDIM, dtype=jnp.int32)).reshape(_NDIM, 1)
    idx_ref[...] = jnp.sum(c * powers, axis=0)


def _make_index_kernel(batch):
    return pl.pallas_call(
        _tc_index_body,
        out_shape=jax.ShapeDtypeStruct((batch,), jnp.int32),
    )


def _make_sc_gather(batch, n_states, out_dim):
    b_per_w = batch // _NUM_WORKERS
    n_chunks = b_per_w // _CHUNK
    pairs_per_chunk = _CHUNK // _LANES
    n_groups = b_per_w // _GRP
    n_vec = out_dim // _LANES
    mesh = plsc.VectorSubcoreMesh(core_axis_name="c", subcore_axis_name="s")

    @functools.partial(
        pl.kernel,
        mesh=mesh,
        compiler_params=pltpu.CompilerParams(use_tc_tiling_on_sc=True),
        out_type=jax.ShapeDtypeStruct((batch, out_dim), jnp.float32),
        scratch_types=[
            [pltpu.VMEM((1, _CHUNK), jnp.int32) for _ in range(n_chunks)],
            pltpu.VMEM((2 * _LANES, _GRP, out_dim), jnp.float32),
            pltpu.VMEM((n_groups, _GRP, out_dim), jnp.float32),
            pltpu.SemaphoreType.DMA,
            pltpu.SemaphoreType.DMA,
        ],
    )
    def sc_gather(idx_hbm, table_hbm, out_hbm, idx_vs, ring_v, out_v, sem,
                  sem_out):
        wid = lax.axis_index("s") * 2 + lax.axis_index("c")
        base = wid * b_per_w
        for k in range(n_chunks):
            pltpu.sync_copy(idx_hbm.at[wid, k], idx_vs[k])

        def slab_copy(b_idx, slot):
            k8 = pl.multiple_of((b_idx >> 3) * _GRP, _GRP)
            return pltpu.make_async_copy(
                table_hbm.at[pl.ds(k8, _GRP)],
                ring_v.at[slot],
                sem,
            )

        def pair_vec(iv, p):
            return iv[0, pl.ds(p * _LANES, _LANES)]

        def fire_pair(iv, p):
            vec = pair_vec(iv, p)
            for bb in range(_LANES):
                slab_copy(vec[bb], (p % 2) * _LANES + bb).start()

        for c in range(n_chunks):
            iv = idx_vs[c]
            fire_pair(iv, 0)

            def body(p, _, iv=iv, c=c):
                nxt = p + 1

                @pl.when(nxt < pairs_per_chunk)
                def _():
                    fire_pair(iv, nxt)

                vec = pair_vec(iv, p)
                qbase = c * 2 * pairs_per_chunk + p * 2
                for bb in range(_LANES):
                    slot = (p % 2) * _LANES + bb
                    slab_copy(0, slot).wait()
                    m = vec[bb] & (_GRP - 1)
                    for v in range(n_vec):
                        acc = jnp.zeros((_LANES,), jnp.float32)
                        for r in range(_GRP):
                            xr = ring_v[slot, r, pl.ds(v * _LANES, _LANES)]
                            acc = jnp.where(m == r, xr, acc)
                        out_v[qbase + bb // _GRP, bb % _GRP,
                              pl.ds(v * _LANES, _LANES)] = acc
                for h in range(2):
                    pltpu.make_async_copy(
                        out_v.at[qbase + h],
                        out_hbm.at[pl.ds(base + (qbase + h) * _GRP, _GRP)],
                        sem_out,
                    ).start()
                return 0

            lax.fori_loop(0, pairs_per_chunk, body, 0)

        for q in range(n_groups):
            pltpu.make_async_copy(
                out_v.at[q],
                out_hbm.at[pl.ds(base + q * _GRP, _GRP)],
                sem_out,
            ).wait()

    return sc_gather


def kernel(preprocessed_states, table):
    batch = preprocessed_states.shape[0]
    n_states, out_dim = table.shape
    b_per_w = batch // _NUM_WORKERS
    idx = _make_index_kernel(batch)(preprocessed_states.T)
    idx4d = idx.reshape(_NUM_WORKERS, b_per_w // _CHUNK, 1, _CHUNK)
    return _make_sc_gather(batch, n_states, out_dim)(idx4d, table)
